# baseline (speedup 1.0000x reference)
import functools

import jax
import jax.numpy as jnp
from jax import lax
from jax.experimental import pallas as pl
from jax.experimental.pallas import tpu as pltpu
from jax.experimental.pallas import tpu_sc as plsc

_N = 2048
_D = 256
_LATENT = 64
_HIDDEN = 512
_K = 16

_BR = 128
_RSC = 512
_RTC = _N - _RSC
_TCH = _RTC // 2

_NW = 32
_SEL_RPW = _TCH // _NW
_CHUNK = 8
_L = 16
_GRP = 16
_NGRP = _N // _L // _GRP

_DSC_RPW = _RSC // _NW
_KCH = 128
_NKCH = 2 * _N // _KCH


def _decode_body(z_ref, w1_ref, b1_ref, w2_ref, b2_ref, out_ref):
    h = jnp.dot(z_ref[...], w1_ref[...], preferred_element_type=jnp.float32)
    h = jnp.maximum(h + b1_ref[...], 0.0)
    out_ref[...] = jnp.dot(h, w2_ref[...], preferred_element_type=jnp.float32) + b2_ref[...]


def _dist_body(a_ref, bt_ref, out_ref, acc_ref):
    def dist_pair(t, a_carry):
        d0 = 2 * t
        acc_ref[...] += jnp.abs(a_carry[:, 0:1] - bt_ref[pl.ds(d0, 1), :])
        a_next = pltpu.roll(a_carry, _D - 1, 1)
        out_ref[...] += jnp.abs(a_next[:, 0:1] - bt_ref[pl.ds(d0 + 1, 1), :])
        return pltpu.roll(a_next, _D - 1, 1)

    a0 = a_ref[...]
    acc_ref[...] = jnp.abs(a0[:, 0:1] - bt_ref[pl.ds(0, 1), :])
    a1 = pltpu.roll(a0, _D - 1, 1)
    out_ref[...] = jnp.abs(a1[:, 0:1] - bt_ref[pl.ds(1, 1), :])
    a2 = pltpu.roll(a1, _D - 1, 1)
    jax.lax.fori_loop(1, _D // 2, dist_pair, a2)
    out_ref[...] += acc_ref[...]


def _sort16(x):
    r = plsc.sort_key_val(x, x)
    return r[0] if isinstance(r, (tuple, list)) else r


def _merge16(a, b):
    return _sort16(jnp.minimum(a, lax.rev(b, (0,))))


def _tree_bottom16(runs):
    runs = [_sort16(r) for r in runs]
    while len(runs) > 1:
        runs = [_merge16(x, y) for x, y in zip(runs[::2], runs[1::2])]
    return runs[0]


def _scores_to_hub(bpos, bneg):
    sp = jnp.sum(1.0 / bpos)
    sn = jnp.sum(1.0 / bneg)
    loss = jnp.maximum((sn - sp) * (1.0 / _K), 0.0)
    return jnp.where(loss < 1.0, 0.5 * loss * loss, loss - 0.5)


def _sc_select_body(dist_hbm, out_hbm, buf, outv):
    c = lax.axis_index("c")
    s = lax.axis_index("s")
    wid = s * 2 + c
    base = wid * _SEL_RPW

    def chunk_step(k, acc):
        pltpu.sync_copy(dist_hbm.at[pl.ds(base + k * _CHUNK, _CHUNK)], buf)

        def row_step(r, acc_r):
            def half_bottom16(col0):
                def grp_step(g, best):
                    grp = _tree_bottom16(
                        [buf[r, pl.ds(col0 + 256 * g + 16 * j, 16)] for j in range(_GRP)])
                    return _merge16(best, grp)

                best0 = _tree_bottom16(
                    [buf[r, pl.ds(col0 + 16 * j, 16)] for j in range(_GRP)])
                return lax.fori_loop(1, _NGRP, grp_step, best0)

            bpos = half_bottom16(0)
            bneg = half_bottom16(_N)
            return acc_r + _scores_to_hub(bpos, bneg)

        return lax.fori_loop(0, _CHUNK, row_step, acc)

    total = lax.fori_loop(0, _SEL_RPW // _CHUNK, chunk_step, jnp.float32(0.0))
    outv[...] = jnp.full((_L,), total, jnp.float32)
    pltpu.sync_copy(outv, out_hbm.at[wid])


def _sc_distsel_body(a_hbm, keys3_hbm, out_hbm, a_buf, kt, bests, outv):
    c = lax.axis_index("c")
    s = lax.axis_index("s")
    wid = s * 2 + c
    base = wid * _DSC_RPW

    pltpu.sync_copy(a_hbm.at[pl.ds(base, _DSC_RPW)], a_buf)
    inf16 = jnp.full((_L,), jnp.inf, jnp.float32)
    for i in range(_DSC_RPW):
        for h in range(2):
            bests[i, pl.ds(h * _L, _L)] = inf16

    def chunk_step(ch, _):
        pltpu.sync_copy(keys3_hbm.at[ch], kt)
        hoff = (ch // (_NKCH // 2)) * _L

        def pair_step(p, __):
            i0 = 2 * p
            i1 = i0 + 1
            accs = [jnp.zeros((_L,), jnp.float32) for _ in range(16)]

            def d_step(t, carry):
                acc = list(carry)
                av0 = a_buf[i0, pl.ds(_L * t, _L)]
                av1 = a_buf[i1, pl.ds(_L * t, _L)]
                for u in range(_L):
                    d = _L * t + u
                    a0 = av0[u]
                    a1 = av1[u]
                    for j in range(8):
                        kv = kt[d, pl.ds(_L * j, _L)]
                        acc[j] = acc[j] + jnp.abs(a0 - kv)
                        acc[8 + j] = acc[8 + j] + jnp.abs(a1 - kv)
                return tuple(acc)

            accs = lax.fori_loop(0, _D // _L, d_step, tuple(accs))
            m0 = _tree_bottom16(list(accs[:8]))
            m1 = _tree_bottom16(list(accs[8:]))
            b0 = bests[i0, pl.ds(hoff, _L)]
            bests[i0, pl.ds(hoff, _L)] = _merge16(b0, m0)
            b1 = bests[i1, pl.ds(hoff, _L)]
            bests[i1, pl.ds(hoff, _L)] = _merge16(b1, m1)
            return 0

        lax.fori_loop(0, _DSC_RPW // 2, pair_step, 0)
        return 0

    lax.fori_loop(0, _NKCH, chunk_step, 0)

    def row_acc(i, acc):
        bpos = bests[i, pl.ds(0, _L)]
        bneg = bests[i, pl.ds(_L, _L)]
        return acc + _scores_to_hub(bpos, bneg)

    total = lax.fori_loop(0, _DSC_RPW, row_acc, jnp.float32(0.0))
    outv[...] = jnp.full((_L,), total, jnp.float32)
    pltpu.sync_copy(outv, out_hbm.at[wid])


def kernel(gt_vals, train_latents, test_latents, W1, b1, W2, b2):
    z = jnp.concatenate([test_latents, train_latents], axis=0)
    b1r = b1.reshape(1, _HIDDEN)
    b2r = b2.reshape(1, _D)

    decoded = pl.pallas_call(
        _decode_body,
        grid=(4,),
        in_specs=[
            pl.BlockSpec((2 * _N // 4, _LATENT), lambda i: (i, 0)),
            pl.BlockSpec((_LATENT, _HIDDEN), lambda i: (0, 0)),
            pl.BlockSpec((1, _HIDDEN), lambda i: (0, 0)),
            pl.BlockSpec((_HIDDEN, _D), lambda i: (0, 0)),
            pl.BlockSpec((1, _D), lambda i: (0, 0)),
        ],
        out_specs=pl.BlockSpec((2 * _N // 4, _D), lambda i: (i, 0)),
        out_shape=jax.ShapeDtypeStruct((2 * _N, _D), jnp.float32),
    )(z, W1, b1r, W2, b2r)

    rec_test = decoded[:_N]
    rec_train = decoded[_N:]
    keys_t = jnp.concatenate([gt_vals, rec_train], axis=0).T
    keys3 = keys_t.reshape(_D, _NKCH, _KCH).transpose(1, 0, 2)

    mesh = plsc.VectorSubcoreMesh(core_axis_name="c", subcore_axis_name="s",
                                  num_cores=2, num_subcores=16)
    sc_params = pltpu.CompilerParams(needs_layout_passes=False)

    sc_distsel = pl.kernel(
        _sc_distsel_body,
        out_type=jax.ShapeDtypeStruct((_NW, _L), jnp.float32),
        mesh=mesh,
        scratch_types=[
            pltpu.VMEM((_DSC_RPW, _D), jnp.float32),
            pltpu.VMEM((_D, _KCH), jnp.float32),
            pltpu.VMEM((_DSC_RPW, 2 * _L), jnp.float32),
            pltpu.VMEM((_L,), jnp.float32),
        ],
        compiler_params=sc_params,
    )
    partials_c = sc_distsel(rec_test[_RTC:], keys3)

    def dist_call(a_half):
        return pl.pallas_call(
            _dist_body,
            grid=(_TCH // _BR,),
            in_specs=[
                pl.BlockSpec((_BR, _D), lambda i: (i, 0)),
                pl.BlockSpec((_D, 2 * _N), lambda i: (0, 0)),
            ],
            out_specs=pl.BlockSpec((_BR, 2 * _N), lambda i: (i, 0)),
            out_shape=jax.ShapeDtypeStruct((_TCH, 2 * _N), jnp.float32),
            scratch_shapes=[pltpu.VMEM((_BR, 2 * _N), jnp.float32)],
        )(a_half, keys_t)

    sc_select = pl.kernel(
        _sc_select_body,
        out_type=jax.ShapeDtypeStruct((_NW, _L), jnp.float32),
        mesh=mesh,
        scratch_types=[
            pltpu.VMEM((_CHUNK, 2 * _N), jnp.float32),
            pltpu.VMEM((_L,), jnp.float32),
        ],
        compiler_params=sc_params,
    )

    dist_a = dist_call(rec_test[:_TCH])
    dist_b = dist_call(rec_test[_TCH:_RTC])
    partials_a = sc_select(dist_a)
    partials_b = sc_select(dist_b)

    total = (jnp.sum(partials_a[:, 0]) + jnp.sum(partials_b[:, 0])
             + jnp.sum(partials_c[:, 0]))
    return (total * (1.0 / _N)).reshape(())


# speedup vs baseline: 3.1696x; 3.1696x over previous
import functools

import jax
import jax.numpy as jnp
from jax import lax
from jax.experimental import pallas as pl
from jax.experimental.pallas import tpu as pltpu
from jax.experimental.pallas import tpu_sc as plsc

_N = 2048
_D = 256
_LATENT = 64
_HIDDEN = 512
_K = 16

_BR = 128
_NB = _N // _BR

_NW = 32
_RPW = _N // _NW
_RPW_HALF = _N // 4 // _NW
_CHUNK = 8
_L = 16
_VPH = 2 * _N // _L // 2
_GRP = 16
_NGRP = _VPH // _GRP


def _decode_body(z_ref, w1_ref, b1_ref, w2_ref, b2_ref, out_ref):
    h = jnp.dot(z_ref[...], w1_ref[...], preferred_element_type=jnp.float32)
    h = jnp.maximum(h + b1_ref[...], 0.0)
    out_ref[...] = jnp.dot(h, w2_ref[...], preferred_element_type=jnp.float32) + b2_ref[...]


def _dist_body(a_ref, bt_ref, out_ref, acc_ref):
    def dist_pair(t, a_carry):
        d0 = 2 * t
        a_col0 = a_carry[:, 0:1]
        b_row0 = bt_ref[pl.ds(d0, 1), :]
        acc_ref[...] += jnp.abs(a_col0 - b_row0)
        a_next = pltpu.roll(a_carry, _D - 1, 1)
        a_col1 = a_next[:, 0:1]
        b_row1 = bt_ref[pl.ds(d0 + 1, 1), :]
        out_ref[...] += jnp.abs(a_col1 - b_row1)
        return pltpu.roll(a_next, _D - 1, 1)

    a0 = a_ref[...]
    acc_ref[...] = jnp.abs(a0[:, 0:1] - bt_ref[pl.ds(0, 1), :])
    a1 = pltpu.roll(a0, _D - 1, 1)
    out_ref[...] = jnp.abs(a1[:, 0:1] - bt_ref[pl.ds(1, 1), :])
    a2 = pltpu.roll(a1, _D - 1, 1)
    jax.lax.fori_loop(1, _D // 2, dist_pair, a2)
    out_ref[...] += acc_ref[...]


def _sort16(x):
    r = plsc.sort_key_val(x, x)
    return r[0] if isinstance(r, (tuple, list)) else r


def _merge16(a, b):
    return _sort16(jnp.minimum(a, lax.rev(b, (0,))))


def _group_bottom16(buf, r, col0):
    runs = [_sort16(buf[r, pl.ds(col0 + 16 * j, 16)]) for j in range(_GRP)]
    while len(runs) > 1:
        runs = [_merge16(x, y) for x, y in zip(runs[::2], runs[1::2])]
    return runs[0]


def _sc_select_body(dist_hbm, out_hbm, buf, outv):
    c = lax.axis_index("c")
    s = lax.axis_index("s")
    wid = s * 2 + c
    base = wid * _RPW_HALF

    def chunk_step(k, acc):
        pltpu.sync_copy(dist_hbm.at[pl.ds(base + k * _CHUNK, _CHUNK)], buf)

        def row_step(r, acc_r):
            def half_bottom16(col0):
                def grp_step(g, best):
                    grp = _group_bottom16(buf, r, col0 + 256 * g)
                    return _merge16(best, grp)

                best0 = _group_bottom16(buf, r, col0)
                return lax.fori_loop(1, _NGRP, grp_step, best0)

            bpos = half_bottom16(0)
            bneg = half_bottom16(_N)
            sp = jnp.sum(1.0 / bpos)
            sn = jnp.sum(1.0 / bneg)
            loss = jnp.maximum((sn - sp) * (1.0 / _K), 0.0)
            hub = jnp.where(loss < 1.0, 0.5 * loss * loss, loss - 0.5)
            return acc_r + hub

        return lax.fori_loop(0, _CHUNK, row_step, acc)

    total = lax.fori_loop(0, _RPW_HALF // _CHUNK, chunk_step, jnp.float32(0.0))
    outv[...] = jnp.full((_L,), total, jnp.float32)
    pltpu.sync_copy(outv, out_hbm.at[wid])


def kernel(gt_vals, train_latents, test_latents, W1, b1, W2, b2):
    z = jnp.concatenate([test_latents, train_latents], axis=0)
    b1r = b1.reshape(1, _HIDDEN)
    b2r = b2.reshape(1, _D)

    decoded = pl.pallas_call(
        _decode_body,
        grid=(4,),
        in_specs=[
            pl.BlockSpec((2 * _N // 4, _LATENT), lambda i: (i, 0)),
            pl.BlockSpec((_LATENT, _HIDDEN), lambda i: (0, 0)),
            pl.BlockSpec((1, _HIDDEN), lambda i: (0, 0)),
            pl.BlockSpec((_HIDDEN, _D), lambda i: (0, 0)),
            pl.BlockSpec((1, _D), lambda i: (0, 0)),
        ],
        out_specs=pl.BlockSpec((2 * _N // 4, _D), lambda i: (i, 0)),
        out_shape=jax.ShapeDtypeStruct((2 * _N, _D), jnp.float32),
    )(z, W1, b1r, W2, b2r)

    rec_test = decoded[:_N]
    rec_train = decoded[_N:]
    keys_t = jnp.concatenate([gt_vals, rec_train], axis=0).T

    half = _N // 4

    def dist_call(a_half):
        return pl.pallas_call(
            _dist_body,
            grid=(_NB // 4,),
            in_specs=[
                pl.BlockSpec((_BR, _D), lambda i: (i, 0)),
                pl.BlockSpec((_D, 2 * _N), lambda i: (0, 0)),
            ],
            out_specs=pl.BlockSpec((_BR, 2 * _N), lambda i: (i, 0)),
            out_shape=jax.ShapeDtypeStruct((half, 2 * _N), jnp.float32),
            scratch_shapes=[pltpu.VMEM((_BR, 2 * _N), jnp.float32)],
        )(a_half, keys_t)

    sc_kernel = pl.kernel(
        _sc_select_body,
        out_type=jax.ShapeDtypeStruct((_NW, _L), jnp.float32),
        mesh=plsc.VectorSubcoreMesh(core_axis_name="c", subcore_axis_name="s",
                                    num_cores=2, num_subcores=16),
        scratch_types=[
            pltpu.VMEM((_CHUNK, 2 * _N), jnp.float32),
            pltpu.VMEM((_L,), jnp.float32),
        ],
        compiler_params=pltpu.CompilerParams(needs_layout_passes=False),
    )

    total = jnp.float32(0.0)
    for q in range(4):
        dist_q = dist_call(rec_test[q * half:(q + 1) * half])
        total = total + jnp.sum(sc_kernel(dist_q)[:, 0])
    return (total * (1.0 / _N)).reshape(())
